# baseline (speedup 1.0000x reference)
import functools

import jax
import jax.numpy as jnp
from jax import lax
from jax.experimental import pallas as pl
from jax.experimental.pallas import tpu as pltpu
from jax.experimental.pallas import tpu_sc as plsc

NS = 16
NC = 2
WIN = 896
SUB = 128
BE = 8192
BN = 800


def _edge_mlp_body(ea1_ref, ea2_ref, w1_ref, b1_ref, out_ref):
    def mlp(ea):
        acc = jnp.broadcast_to(b1_ref[...], (ea.shape[0], 64))
        for k in range(4):
            acc = acc + ea[:, k:k + 1] * w1_ref[k:k + 1, :]
        return jnp.maximum(acc, 0.0)
    out_ref[...] = jnp.concatenate([mlp(ea1_ref[...]), mlp(ea2_ref[...])],
                                   axis=1)


def _scatter_body(n_rows_pt, n_wins, row_ref, a_ref, z2_ref, o2_ref,
                  acc_out, cnt_out, idx2, vals, acc_sp):
    core = lax.axis_index("c")
    sid = lax.axis_index("s")
    r0 = sid * n_rows_pt
    nz = n_rows_pt // WIN
    ept = n_wins * WIN

    for p in range(3):
        pltpu.sync_copy(z2_ref, vals)
        for kk in range(nz):
            pltpu.sync_copy(vals, acc_sp.at[pl.ds(r0 + kk * WIN, WIN), :])
        plsc.subcore_barrier()

        if p < 2:
            fc = core * 2 + p

            def win(w, carry):
                e0 = sid * ept + w * WIN
                g0 = sid * (ept // SUB) + w * (WIN // SUB)
                pltpu.sync_copy(row_ref.at[pl.ds(g0, WIN // SUB), :], idx2)
                pltpu.sync_copy(a_ref.at[pl.ds(e0, WIN), pl.ds(fc * 16, 16)],
                                vals)
                for j in range(WIN // SUB):
                    pltpu.sync_copy(vals.at[pl.ds(j * SUB, SUB), :],
                                    acc_sp.at[idx2.at[j]], add=True)
                return carry

            lax.fori_loop(0, n_wins, win, 0)
        else:
            pltpu.sync_copy(o2_ref, vals)
            half = sid * (ept // 2) + core * (NS * ept // 2)

            def cwin(w, carry):
                g0 = (half + w * WIN) // SUB
                pltpu.sync_copy(row_ref.at[pl.ds(g0, WIN // SUB), :], idx2)
                for j in range(WIN // SUB):
                    pltpu.sync_copy(vals.at[pl.ds(j * SUB, SUB), :],
                                    acc_sp.at[idx2.at[j]], add=True)
                return carry

            lax.fori_loop(0, n_wins // 2, cwin, 0)
        plsc.subcore_barrier()

        for kk in range(nz):
            rr = r0 + kk * WIN
            pltpu.sync_copy(acc_sp.at[pl.ds(rr, WIN), :], vals)
            if p < 2:
                pltpu.sync_copy(vals, acc_out.at[core * 2 + p,
                                                 pl.ds(rr, WIN), :])
            else:
                pltpu.sync_copy(vals, cnt_out.at[core, pl.ds(rr, WIN), :])
        plsc.subcore_barrier()


def _node_mlp_body(acc_ref, cnt_ref, w23_ref, bbf_ref, b3_ref, w4_ref, b4_ref,
                   out_ref):
    total = cnt_ref[...]
    denom = jnp.maximum(total, 1.0)
    f = jnp.where(total > 0.0, 1.0, 0.0)
    mean_a = jnp.concatenate(
        [acc_ref[0], acc_ref[1], acc_ref[2], acc_ref[3]], axis=1) / denom
    h2 = jnp.dot(mean_a, w23_ref[...], preferred_element_type=jnp.float32)
    h2 = jnp.maximum(h2 + f * bbf_ref[...] + b3_ref[...], 0.0)
    out = jnp.dot(h2, w4_ref[...], preferred_element_type=jnp.float32)
    out_ref[...] = out + b4_ref[...]


def kernel(x, edge_index, edge_attr, u, batch, W1, b1, W2, b2, W3, b3, W4, b4):
    N = x.shape[0]
    E = edge_attr.shape[0]
    f32 = jnp.float32

    n_rows_pt = -(-(-(-N // NS)) // WIN) * WIN
    Np = NS * n_rows_pt
    n_wins = (E + NS * WIN - 1) // (NS * WIN)
    Ep = NS * n_wins * WIN
    pad_e = Ep - E
    pad_rows = Np - N

    row = edge_index[0]
    pad_idx = N + (jnp.arange(pad_e, dtype=jnp.int32) % pad_rows)
    row_flat = jnp.concatenate([row, pad_idx])
    row_p = jnp.stack([row_flat[:Ep // 2], row_flat[Ep // 2:]],
                      axis=1).reshape(Ep // SUB, SUB)
    ea_p = jnp.concatenate([edge_attr, jnp.zeros((pad_e, 4), f32)])

    nb = Ep // BE
    a = pl.pallas_call(
        _edge_mlp_body,
        grid=(nb,),
        in_specs=[
            pl.BlockSpec((BE // 2, 4), lambda i: (i, 0)),
            pl.BlockSpec((BE // 2, 4), lambda i: (i + Ep // BE, 0)),
            pl.BlockSpec((4, 64), lambda i: (0, 0)),
            pl.BlockSpec((1, 64), lambda i: (0, 0)),
        ],
        out_specs=pl.BlockSpec((BE // 2, 128), lambda i: (i, 0)),
        out_shape=jax.ShapeDtypeStruct((Ep // 2, 128), f32),
    )(ea_p, ea_p, W1, b1.reshape(1, 64))
    a = a.reshape(Ep, 64)

    z2 = jnp.zeros((WIN, 16), f32)
    o2 = jnp.ones((WIN, 16), f32)
    mesh = plsc.VectorSubcoreMesh(core_axis_name="c", subcore_axis_name="s")
    sc_fn = pl.kernel(
        functools.partial(_scatter_body, n_rows_pt, n_wins),
        out_type=(jax.ShapeDtypeStruct((4, Np, 16), f32),
                  jax.ShapeDtypeStruct((NC, Np, 16), f32)),
        mesh=mesh,
        compiler_params=pltpu.CompilerParams(use_tc_tiling_on_sc=False),
        scratch_types=[
            pltpu.VMEM((WIN // SUB, SUB), jnp.int32),
            pltpu.VMEM((WIN, 16), f32),
            pltpu.VMEM_SHARED((Np, 16), f32),
        ],
    )
    acc, cnt = sc_fn(row_p, a, z2, o2)
    cnt_t = (cnt[0, :, 0] + cnt[1, :, 0]).reshape(Np, 1)

    W23 = W2 @ W3
    bbf = (b2 @ W3).reshape(1, 256)
    out = pl.pallas_call(
        _node_mlp_body,
        grid=(N // BN,),
        in_specs=[
            pl.BlockSpec((4, BN, 16), lambda i: (0, i, 0)),
            pl.BlockSpec((BN, 1), lambda i: (i, 0)),
            pl.BlockSpec((64, 256), lambda i: (0, 0)),
            pl.BlockSpec((1, 256), lambda i: (0, 0)),
            pl.BlockSpec((1, 256), lambda i: (0, 0)),
            pl.BlockSpec((256, 512), lambda i: (0, 0)),
            pl.BlockSpec((1, 512), lambda i: (0, 0)),
        ],
        out_specs=pl.BlockSpec((BN, 512), lambda i: (i, 0)),
        out_shape=jax.ShapeDtypeStruct((N, 512), f32),
    )(acc, cnt_t, W23, bbf, b3.reshape(1, 256), W4, b4.reshape(1, 512))
    return out


# speedup vs baseline: 2.4531x; 2.4531x over previous
import functools

import jax
import jax.numpy as jnp
from jax import lax
from jax.experimental import pallas as pl
from jax.experimental.pallas import tpu as pltpu
from jax.experimental.pallas import tpu_sc as plsc

NS = 16
NC = 2
WIN = 896
SUB = 128
BE = 8192
BN = 800


def _scatter_body(n_rows_pt, n_wins, row_ref, ea_ref, w1_ref, b1_ref,
                  z2_ref, o2_ref, acc_out, cnt_out, idx2, vals, eav, w1v, b1v,
                  acc_sp):
    core = lax.axis_index("c")
    sid = lax.axis_index("s")
    r0 = sid * n_rows_pt
    nz = n_rows_pt // WIN
    ept = n_wins * WIN
    nsub = WIN // SUB
    pltpu.sync_copy(w1_ref, w1v)
    pltpu.sync_copy(b1_ref, b1v)
    iota16 = lax.iota(jnp.int32, 16)

    for p in range(3):
        pltpu.sync_copy(z2_ref, vals)
        for kk in range(nz):
            pltpu.sync_copy(vals, acc_sp.at[pl.ds(r0 + kk * WIN, WIN), :])
        plsc.subcore_barrier()

        if p < 2:
            fc = core * 2 + p
            fcb = fc * 16

            def win(w, carry):
                g0 = sid * (ept // SUB) + w * nsub
                pltpu.sync_copy(row_ref.at[pl.ds(g0, nsub), :], idx2)
                for k in range(4):
                    pltpu.sync_copy(ea_ref.at[k, pl.ds(g0, nsub), :],
                                    eav.at[pl.ds(k * nsub, nsub), :])
                ws = [w1v[k, pl.ds(fcb, 16)] for k in range(4)]
                bs = b1v[0, pl.ds(fcb, 16)]

                def rowloop(r, carry2):
                    for gg in range(SUB // 16):
                        ev = [eav[k * nsub + r, pl.ds(gg * 16, 16)]
                              for k in range(4)]
                        eg = r * SUB + gg * 16
                        for f in range(16):
                            acc = jnp.full((16,), bs[f], dtype=jnp.float32)
                            for k in range(4):
                                acc = acc + ev[k] * ws[k][f]
                            af = jnp.maximum(acc, 0.0)
                            plsc.store_scatter(
                                vals, [eg + iota16,
                                       jnp.full((16,), f, dtype=jnp.int32)],
                                af)
                    return carry2

                lax.fori_loop(0, nsub, rowloop, 0)
                for j in range(nsub):
                    pltpu.sync_copy(vals.at[pl.ds(j * SUB, SUB), :],
                                    acc_sp.at[idx2.at[j]], add=True)
                return carry

            lax.fori_loop(0, n_wins, win, 0)
        else:
            pltpu.sync_copy(o2_ref, vals)
            half = sid * (ept // 2) + core * (NS * ept // 2)

            def cwin(w, carry):
                g0 = (half + w * WIN) // SUB
                pltpu.sync_copy(row_ref.at[pl.ds(g0, WIN // SUB), :], idx2)
                for j in range(WIN // SUB):
                    pltpu.sync_copy(vals.at[pl.ds(j * SUB, SUB), :],
                                    acc_sp.at[idx2.at[j]], add=True)
                return carry

            lax.fori_loop(0, n_wins // 2, cwin, 0)
        plsc.subcore_barrier()

        for kk in range(nz):
            rr = r0 + kk * WIN
            pltpu.sync_copy(acc_sp.at[pl.ds(rr, WIN), :], vals)
            if p < 2:
                pltpu.sync_copy(vals, acc_out.at[core * 2 + p,
                                                 pl.ds(rr, WIN), :])
            else:
                pltpu.sync_copy(vals, cnt_out.at[core, pl.ds(rr, WIN), :])
        plsc.subcore_barrier()


def _node_mlp_body(acc_ref, cnt_ref, w23_ref, bbf_ref, b3_ref, w4_ref, b4_ref,
                   out_ref):
    total = cnt_ref[...]
    denom = jnp.maximum(total, 1.0)
    f = jnp.where(total > 0.0, 1.0, 0.0)
    mean_a = jnp.concatenate(
        [acc_ref[0], acc_ref[1], acc_ref[2], acc_ref[3]], axis=1) / denom
    h2 = jnp.dot(mean_a, w23_ref[...], preferred_element_type=jnp.float32)
    h2 = jnp.maximum(h2 + f * bbf_ref[...] + b3_ref[...], 0.0)
    out = jnp.dot(h2, w4_ref[...], preferred_element_type=jnp.float32)
    out_ref[...] = out + b4_ref[...]


def kernel(x, edge_index, edge_attr, u, batch, W1, b1, W2, b2, W3, b3, W4, b4):
    N = x.shape[0]
    E = edge_attr.shape[0]
    f32 = jnp.float32

    n_rows_pt = -(-(-(-N // NS)) // WIN) * WIN
    Np = NS * n_rows_pt
    n_wins = (E + NS * WIN - 1) // (NS * WIN)
    Ep = NS * n_wins * WIN
    pad_e = Ep - E
    pad_rows = Np - N

    row = edge_index[0]
    pad_idx = N + (jnp.arange(pad_e, dtype=jnp.int32) % pad_rows)
    row_p = jnp.concatenate([row, pad_idx]).reshape(Ep // SUB, SUB)
    ea_p = jnp.concatenate([edge_attr, jnp.zeros((pad_e, 4), f32)])

    ea_t = ea_p.T.reshape(4, Ep // SUB, SUB)

    z2 = jnp.zeros((WIN, 16), f32)
    o2 = jnp.ones((WIN, 16), f32)
    mesh = plsc.VectorSubcoreMesh(core_axis_name="c", subcore_axis_name="s")
    sc_fn = pl.kernel(
        functools.partial(_scatter_body, n_rows_pt, n_wins),
        out_type=(jax.ShapeDtypeStruct((4, Np, 16), f32),
                  jax.ShapeDtypeStruct((NC, Np, 16), f32)),
        mesh=mesh,
        compiler_params=pltpu.CompilerParams(use_tc_tiling_on_sc=False,
                                             needs_layout_passes=False),
        scratch_types=[
            pltpu.VMEM((WIN // SUB, SUB), jnp.int32),
            pltpu.VMEM((WIN, 16), f32),
            pltpu.VMEM((4 * (WIN // SUB), SUB), f32),
            pltpu.VMEM((4, 64), f32),
            pltpu.VMEM((1, 64), f32),
            pltpu.VMEM_SHARED((Np, 16), f32),
        ],
    )
    acc, cnt = sc_fn(row_p, ea_t, W1, b1.reshape(1, 64), z2, o2)
    cnt_t = (cnt[0, :, 0] + cnt[1, :, 0]).reshape(Np, 1)

    W23 = W2 @ W3
    bbf = (b2 @ W3).reshape(1, 256)
    out = pl.pallas_call(
        _node_mlp_body,
        grid=(N // BN,),
        in_specs=[
            pl.BlockSpec((4, BN, 16), lambda i: (0, i, 0)),
            pl.BlockSpec((BN, 1), lambda i: (i, 0)),
            pl.BlockSpec((64, 256), lambda i: (0, 0)),
            pl.BlockSpec((1, 256), lambda i: (0, 0)),
            pl.BlockSpec((1, 256), lambda i: (0, 0)),
            pl.BlockSpec((256, 512), lambda i: (0, 0)),
            pl.BlockSpec((1, 512), lambda i: (0, 0)),
        ],
        out_specs=pl.BlockSpec((BN, 512), lambda i: (i, 0)),
        out_shape=jax.ShapeDtypeStruct((N, 512), f32),
    )(acc, cnt_t, W23, bbf, b3.reshape(1, 256), W4, b4.reshape(1, 512))
    return out


# speedup vs baseline: 2.9622x; 1.2075x over previous
import functools

import jax
import jax.numpy as jnp
from jax import lax
from jax.experimental import pallas as pl
from jax.experimental.pallas import tpu as pltpu
from jax.experimental.pallas import tpu_sc as plsc

NS = 16
NC = 2
WIN = 512
SUB = 128
ZC = 448
BE = 8192
BN = 800


def _scatter_body(n_rows_pt, n_wins, row_ref, ea_ref, w1_ref, b1_ref,
                  z2_ref, o2_ref, acc_out, cnt_out, idx2, vals, eav, w1v, b1v,
                  acc_sp, sem):
    core = lax.axis_index("c")
    sid = lax.axis_index("s")
    r0 = sid * n_rows_pt
    nz = n_rows_pt // ZC
    ept = n_wins * WIN
    nsub = WIN // SUB
    pltpu.sync_copy(w1_ref, w1v)
    pltpu.sync_copy(b1_ref, b1v)
    iota16 = lax.iota(jnp.int32, 16)

    def drain_one_window():
        pltpu.make_async_copy(z2_ref, vals.at[0], sem).wait()

    for p in range(3):
        pltpu.sync_copy(z2_ref.at[pl.ds(0, ZC), :], vals.at[0, pl.ds(0, ZC), :])
        for kk in range(nz):
            pltpu.sync_copy(vals.at[0, pl.ds(0, ZC), :],
                            acc_sp.at[pl.ds(r0 + kk * ZC, ZC), :])
        plsc.subcore_barrier()

        if p < 2:
            fc = core * 2 + p
            fcb = fc * 16

            def win(w, carry):
                b = w % 2
                pl.when(w >= 2)(drain_one_window)
                g0 = sid * (ept // SUB) + w * nsub
                pltpu.sync_copy(row_ref.at[pl.ds(g0, nsub), :], idx2.at[b])
                pltpu.sync_copy(ea_ref.at[pl.ds(g0, nsub), :, :], eav)
                ws = [w1v[k, pl.ds(fcb, 16)] for k in range(4)]
                bs = b1v[0, pl.ds(fcb, 16)]
                bvec = jnp.full((16,), b, dtype=jnp.int32)

                def rowloop(r, carry2):
                    for gg in range(SUB // 16):
                        ev = [eav[r, k, pl.ds(gg * 16, 16)]
                              for k in range(4)]
                        eg = r * SUB + gg * 16
                        for f in range(16):
                            acc = jnp.full((16,), bs[f], dtype=jnp.float32)
                            for k in range(4):
                                acc = acc + ev[k] * ws[k][f]
                            af = jnp.maximum(acc, 0.0)
                            plsc.store_scatter(
                                vals, [bvec, eg + iota16,
                                       jnp.full((16,), f, dtype=jnp.int32)],
                                af)
                    return carry2

                lax.fori_loop(0, nsub, rowloop, 0)
                for j in range(nsub):
                    pltpu.async_copy(vals.at[b, pl.ds(j * SUB, SUB), :],
                                     acc_sp.at[idx2.at[b, j]], sem, add=True)
                return carry

            lax.fori_loop(0, n_wins, win, 0)
        else:
            pltpu.sync_copy(o2_ref, vals.at[0])
            pltpu.sync_copy(o2_ref, vals.at[1])
            half = sid * (ept // 2) + core * (NS * ept // 2)

            def cwin(w, carry):
                b = w % 2
                pl.when(w >= 2)(drain_one_window)
                g0 = (half + w * WIN) // SUB
                pltpu.sync_copy(row_ref.at[pl.ds(g0, nsub), :], idx2.at[b])
                for j in range(nsub):
                    pltpu.async_copy(vals.at[b, pl.ds(j * SUB, SUB), :],
                                     acc_sp.at[idx2.at[b, j]], sem, add=True)
                return carry

            lax.fori_loop(0, n_wins // 2, cwin, 0)
        drain_one_window()
        drain_one_window()
        plsc.subcore_barrier()

        for kk in range(nz):
            rr = r0 + kk * ZC
            pltpu.sync_copy(acc_sp.at[pl.ds(rr, ZC), :],
                            vals.at[0, pl.ds(0, ZC), :])
            if p < 2:
                pltpu.sync_copy(vals.at[0, pl.ds(0, ZC), :],
                                acc_out.at[core * 2 + p, pl.ds(rr, ZC), :])
            else:
                pltpu.sync_copy(vals.at[0, pl.ds(0, ZC), :],
                                cnt_out.at[core, pl.ds(rr, ZC), :])
        plsc.subcore_barrier()


def _node_mlp_body(acc_ref, cnt_ref, w23_ref, bbf_ref, b3_ref, w4_ref, b4_ref,
                   out_ref):
    total = cnt_ref[...]
    denom = jnp.maximum(total, 1.0)
    f = jnp.where(total > 0.0, 1.0, 0.0)
    mean_a = jnp.concatenate(
        [acc_ref[0], acc_ref[1], acc_ref[2], acc_ref[3]], axis=1) / denom
    h2 = jnp.dot(mean_a, w23_ref[...], preferred_element_type=jnp.float32)
    h2 = jnp.maximum(h2 + f * bbf_ref[...] + b3_ref[...], 0.0)
    out = jnp.dot(h2, w4_ref[...], preferred_element_type=jnp.float32)
    out_ref[...] = out + b4_ref[...]


def kernel(x, edge_index, edge_attr, u, batch, W1, b1, W2, b2, W3, b3, W4, b4):
    N = x.shape[0]
    E = edge_attr.shape[0]
    f32 = jnp.float32

    n_rows_pt = -(-(-(-N // NS)) // ZC) * ZC
    Np = NS * n_rows_pt
    n_wins = (E + NS * WIN - 1) // (NS * WIN)
    Ep = NS * n_wins * WIN
    pad_e = Ep - E
    pad_rows = Np - N

    row = edge_index[0]
    pad_idx = N + (jnp.arange(pad_e, dtype=jnp.int32) % pad_rows)
    row_p = jnp.concatenate([row, pad_idx]).reshape(Ep // SUB, SUB)
    ea_p = jnp.concatenate([edge_attr, jnp.zeros((pad_e, 4), f32)])

    ea_t = ea_p.reshape(Ep // SUB, SUB, 4).transpose(0, 2, 1)

    z2 = jnp.zeros((WIN, 16), f32)
    o2 = jnp.ones((WIN, 16), f32)
    mesh = plsc.VectorSubcoreMesh(core_axis_name="c", subcore_axis_name="s")
    sc_fn = pl.kernel(
        functools.partial(_scatter_body, n_rows_pt, n_wins),
        out_type=(jax.ShapeDtypeStruct((4, Np, 16), f32),
                  jax.ShapeDtypeStruct((NC, Np, 16), f32)),
        mesh=mesh,
        compiler_params=pltpu.CompilerParams(use_tc_tiling_on_sc=False,
                                             needs_layout_passes=False),
        scratch_types=[
            pltpu.VMEM((2, WIN // SUB, SUB), jnp.int32),
            pltpu.VMEM((2, WIN, 16), f32),
            pltpu.VMEM((WIN // SUB, 4, SUB), f32),
            pltpu.VMEM((4, 64), f32),
            pltpu.VMEM((1, 64), f32),
            pltpu.VMEM_SHARED((Np, 16), f32),
            pltpu.SemaphoreType.DMA,
        ],
    )
    acc, cnt = sc_fn(row_p, ea_t, W1, b1.reshape(1, 64), z2, o2)
    cnt_t = (cnt[0, :, 0] + cnt[1, :, 0]).reshape(Np, 1)

    W23 = W2 @ W3
    bbf = (b2 @ W3).reshape(1, 256)
    out = pl.pallas_call(
        _node_mlp_body,
        grid=(N // BN,),
        in_specs=[
            pl.BlockSpec((4, BN, 16), lambda i: (0, i, 0)),
            pl.BlockSpec((BN, 1), lambda i: (i, 0)),
            pl.BlockSpec((64, 256), lambda i: (0, 0)),
            pl.BlockSpec((1, 256), lambda i: (0, 0)),
            pl.BlockSpec((1, 256), lambda i: (0, 0)),
            pl.BlockSpec((256, 512), lambda i: (0, 0)),
            pl.BlockSpec((1, 512), lambda i: (0, 0)),
        ],
        out_specs=pl.BlockSpec((BN, 512), lambda i: (i, 0)),
        out_shape=jax.ShapeDtypeStruct((N, 512), f32),
    )(acc, cnt_t, W23, bbf, b3.reshape(1, 256), W4, b4.reshape(1, 512))
    return out


# speedup vs baseline: 3.1599x; 1.0667x over previous
import functools

import jax
import jax.numpy as jnp
from jax import lax
from jax.experimental import pallas as pl
from jax.experimental.pallas import tpu as pltpu
from jax.experimental.pallas import tpu_sc as plsc

NS = 16
NC = 2
WIN = 512
SUB = 128
ZC = 448
BE = 8192
BN = 800


def _scatter_body(n_rows_pt, n_wins, row_ref, ea_ref, w1_ref, b1_ref,
                  z2_ref, o2_ref, acc_out, cnt_out, idx2, vals, eav, w1v, b1v,
                  acc_sp, sem):
    core = lax.axis_index("c")
    sid = lax.axis_index("s")
    r0 = sid * n_rows_pt
    nz = n_rows_pt // ZC
    ept = n_wins * WIN
    nsub = WIN // SUB
    pltpu.sync_copy(w1_ref, w1v)
    pltpu.sync_copy(b1_ref, b1v)
    iota16 = lax.iota(jnp.int32, 16)

    def drain_one_window():
        pltpu.make_async_copy(z2_ref, vals.at[0], sem).wait()

    for p in range(3):
        pltpu.sync_copy(z2_ref.at[pl.ds(0, ZC), :], vals.at[0, pl.ds(0, ZC), :])
        for kk in range(nz):
            pltpu.sync_copy(vals.at[0, pl.ds(0, ZC), :],
                            acc_sp.at[pl.ds(r0 + kk * ZC, ZC), :])
        plsc.subcore_barrier()

        if p < 2:
            fc = core * 2 + p
            fcb = fc * 16

            def win(w, carry):
                b = w % 2
                pl.when(w >= 2)(drain_one_window)
                g0 = sid * (ept // SUB) + w * nsub
                pltpu.sync_copy(row_ref.at[pl.ds(g0, nsub), :], idx2.at[b])
                pltpu.sync_copy(ea_ref.at[pl.ds(g0, nsub), :, :], eav)
                ws = [w1v[k, pl.ds(fcb, 16)] for k in range(4)]
                bs = b1v[0, pl.ds(fcb, 16)]
                bvec = jnp.full((16,), b, dtype=jnp.int32)

                def make_rowloop(f0):
                    wspl = [[jnp.full((16,), ws[k][f0 + f], dtype=jnp.float32)
                             for k in range(4)] for f in range(8)]
                    bspl = [jnp.full((16,), bs[f0 + f], dtype=jnp.float32)
                            for f in range(8)]

                    def rowloop(r, carry2):
                        for gg in range(SUB // 16):
                            ev = [eav[r, k, pl.ds(gg * 16, 16)]
                                  for k in range(4)]
                            eg = r * SUB + gg * 16
                            for f in range(8):
                                acc = bspl[f]
                                for k in range(4):
                                    acc = acc + ev[k] * wspl[f][k]
                                af = jnp.maximum(acc, 0.0)
                                plsc.store_scatter(
                                    vals,
                                    [bvec, eg + iota16,
                                     jnp.full((16,), f0 + f,
                                              dtype=jnp.int32)],
                                    af)
                        return carry2

                    return rowloop

                lax.fori_loop(0, nsub, make_rowloop(0), 0)
                lax.fori_loop(0, nsub, make_rowloop(8), 0)
                for j in range(nsub):
                    pltpu.async_copy(vals.at[b, pl.ds(j * SUB, SUB), :],
                                     acc_sp.at[idx2.at[b, j]], sem, add=True)
                return carry

            lax.fori_loop(0, n_wins, win, 0)
        else:
            pltpu.sync_copy(o2_ref, vals.at[0])
            pltpu.sync_copy(o2_ref, vals.at[1])
            half = sid * (ept // 2) + core * (NS * ept // 2)

            def cwin(w, carry):
                b = w % 2
                pl.when(w >= 2)(drain_one_window)
                g0 = (half + w * WIN) // SUB
                pltpu.sync_copy(row_ref.at[pl.ds(g0, nsub), :], idx2.at[b])
                for j in range(nsub):
                    pltpu.async_copy(vals.at[b, pl.ds(j * SUB, SUB), :],
                                     acc_sp.at[idx2.at[b, j]], sem, add=True)
                return carry

            lax.fori_loop(0, n_wins // 2, cwin, 0)
        drain_one_window()
        drain_one_window()
        plsc.subcore_barrier()

        for kk in range(nz):
            rr = r0 + kk * ZC
            pltpu.sync_copy(acc_sp.at[pl.ds(rr, ZC), :],
                            vals.at[0, pl.ds(0, ZC), :])
            if p < 2:
                pltpu.sync_copy(vals.at[0, pl.ds(0, ZC), :],
                                acc_out.at[core * 2 + p, pl.ds(rr, ZC), :])
            else:
                pltpu.sync_copy(vals.at[0, pl.ds(0, ZC), :],
                                cnt_out.at[core, pl.ds(rr, ZC), :])
        plsc.subcore_barrier()


def _node_mlp_body(acc_ref, cnt_ref, w23_ref, bbf_ref, b3_ref, w4_ref, b4_ref,
                   out_ref):
    total = cnt_ref[...]
    denom = jnp.maximum(total, 1.0)
    f = jnp.where(total > 0.0, 1.0, 0.0)
    mean_a = jnp.concatenate(
        [acc_ref[0], acc_ref[1], acc_ref[2], acc_ref[3]], axis=1) / denom
    h2 = jnp.dot(mean_a, w23_ref[...], preferred_element_type=jnp.float32)
    h2 = jnp.maximum(h2 + f * bbf_ref[...] + b3_ref[...], 0.0)
    out = jnp.dot(h2, w4_ref[...], preferred_element_type=jnp.float32)
    out_ref[...] = out + b4_ref[...]


def kernel(x, edge_index, edge_attr, u, batch, W1, b1, W2, b2, W3, b3, W4, b4):
    N = x.shape[0]
    E = edge_attr.shape[0]
    f32 = jnp.float32

    n_rows_pt = -(-(-(-N // NS)) // ZC) * ZC
    Np = NS * n_rows_pt
    n_wins = (E + NS * WIN - 1) // (NS * WIN)
    Ep = NS * n_wins * WIN
    pad_e = Ep - E
    pad_rows = Np - N

    row = edge_index[0]
    pad_idx = N + (jnp.arange(pad_e, dtype=jnp.int32) % pad_rows)
    row_p = jnp.concatenate([row, pad_idx]).reshape(Ep // SUB, SUB)
    ea_p = jnp.concatenate([edge_attr, jnp.zeros((pad_e, 4), f32)])

    ea_t = ea_p.reshape(Ep // SUB, SUB, 4).transpose(0, 2, 1)

    z2 = jnp.zeros((WIN, 16), f32)
    o2 = jnp.ones((WIN, 16), f32)
    mesh = plsc.VectorSubcoreMesh(core_axis_name="c", subcore_axis_name="s")
    sc_fn = pl.kernel(
        functools.partial(_scatter_body, n_rows_pt, n_wins),
        out_type=(jax.ShapeDtypeStruct((4, Np, 16), f32),
                  jax.ShapeDtypeStruct((NC, Np, 16), f32)),
        mesh=mesh,
        compiler_params=pltpu.CompilerParams(use_tc_tiling_on_sc=False,
                                             needs_layout_passes=False),
        scratch_types=[
            pltpu.VMEM((2, WIN // SUB, SUB), jnp.int32),
            pltpu.VMEM((2, WIN, 16), f32),
            pltpu.VMEM((WIN // SUB, 4, SUB), f32),
            pltpu.VMEM((4, 64), f32),
            pltpu.VMEM((1, 64), f32),
            pltpu.VMEM_SHARED((Np, 16), f32),
            pltpu.SemaphoreType.DMA,
        ],
    )
    acc, cnt = sc_fn(row_p, ea_t, W1, b1.reshape(1, 64), z2, o2)
    cnt_t = (cnt[0, :, 0] + cnt[1, :, 0]).reshape(Np, 1)

    W23 = W2 @ W3
    bbf = (b2 @ W3).reshape(1, 256)
    out = pl.pallas_call(
        _node_mlp_body,
        grid=(N // BN,),
        in_specs=[
            pl.BlockSpec((4, BN, 16), lambda i: (0, i, 0)),
            pl.BlockSpec((BN, 1), lambda i: (i, 0)),
            pl.BlockSpec((64, 256), lambda i: (0, 0)),
            pl.BlockSpec((1, 256), lambda i: (0, 0)),
            pl.BlockSpec((1, 256), lambda i: (0, 0)),
            pl.BlockSpec((256, 512), lambda i: (0, 0)),
            pl.BlockSpec((1, 512), lambda i: (0, 0)),
        ],
        out_specs=pl.BlockSpec((BN, 512), lambda i: (i, 0)),
        out_shape=jax.ShapeDtypeStruct((N, 512), f32),
    )(acc, cnt_t, W23, bbf, b3.reshape(1, 256), W4, b4.reshape(1, 512))
    return out
